# Initial kernel scaffold; baseline (speedup 1.0000x reference)
import jax
import jax.numpy as jnp
from jax.experimental import pallas as pl


def kernel(score_map, offset_map):
    raise NotImplementedError("write your pallas kernel here")


# speedup vs baseline: 1.9306x; 1.9306x over previous
import functools

import jax
import jax.numpy as jnp
from jax import lax
from jax.experimental import pallas as pl
from jax.experimental.pallas import tpu as pltpu
from jax.experimental.pallas import tpu_sc as plsc

_STRIDE = 4
_BS = 32
_NPTS = 17
_H = 128
_W = 128
_FLAT = _H * _W
_NCH = 2 * _NPTS
_NPTS_PAD = 24
_PAD = 48


def _tc_argmax_body(s_ref, o_ref):
    s = s_ref[0]
    m = jnp.max(s, axis=1, keepdims=True)
    iota = lax.broadcasted_iota(jnp.int32, (_NPTS, _FLAT), 1)
    cand = jnp.where(s == m, iota, jnp.int32(_FLAT))
    idx = jnp.min(cand, axis=1, keepdims=True)
    idx = jnp.pad(idx, ((0, _NPTS_PAD - _NPTS), (0, 0)))
    o_ref[0] = jnp.broadcast_to(idx, (_NPTS_PAD, 128))


def _tc_argmax(score3):
    return pl.pallas_call(
        _tc_argmax_body,
        grid=(_BS,),
        in_specs=[pl.BlockSpec((1, _NPTS, _FLAT), lambda i: (i, 0, 0))],
        out_specs=pl.BlockSpec((1, _NPTS_PAD, 128), lambda i: (i, 0, 0)),
        out_shape=jax.ShapeDtypeStruct((_BS, _NPTS_PAD, 128), jnp.int32),
    )(score3)


def _sc_gather_body(idx_hbm, offtab_hbm, out_hbm,
                    ptrs_v, iv_v, offidx_v, off_v, out_v, sem):
    b = lax.axis_index("s") * 2 + lax.axis_index("c")
    for base in (0, 16, 32):
        jv = lax.iota(jnp.int32, 16) + base
        ptrs_v[pl.ds(base, 16)] = (b * _NPTS_PAD + (jv >> 1)) * 128
    pltpu.async_copy(idx_hbm.at[ptrs_v], iv_v, sem).wait()
    for base in (0, 16, 32):
        jv = lax.iota(jnp.int32, 16) + base
        iv = iv_v[pl.ds(base, 16)]
        yv = iv >> 7
        xv = iv & (_W - 1)
        oidx = ((b * _NCH + jv) * _H + yv) * _W + xv
        offidx_v[pl.ds(base, 16)] = jnp.where(jv < _NCH, oidx, 0)
    pltpu.async_copy(offtab_hbm.at[offidx_v], off_v, sem).wait()
    for base in (0, 16, 32):
        jv = lax.iota(jnp.int32, 16) + base
        iv = iv_v[pl.ds(base, 16)]
        yv = iv >> 7
        xv = iv & (_W - 1)
        coarse = jnp.where((jv & 1) == 0, yv, xv).astype(jnp.float32)
        off = off_v[pl.ds(base, 16)]
        out_v[pl.ds(base, 16)] = (coarse + off) * float(_STRIDE)
    pltpu.sync_copy(out_v, out_hbm.at[pl.ds(b * _PAD, _PAD)])


def _sc_gather(idx_flat, offset_flat):
    mesh = plsc.VectorSubcoreMesh(core_axis_name="c", subcore_axis_name="s")
    f = functools.partial(
        pl.kernel,
        mesh=mesh,
        out_type=jax.ShapeDtypeStruct((_BS * _PAD,), jnp.float32),
        scratch_types=[
            pltpu.VMEM((_PAD,), jnp.int32),
            pltpu.VMEM((_PAD,), jnp.int32),
            pltpu.VMEM((_PAD,), jnp.int32),
            pltpu.VMEM((_PAD,), jnp.float32),
            pltpu.VMEM((_PAD,), jnp.float32),
            pltpu.SemaphoreType.DMA,
        ],
    )(_sc_gather_body)
    return f(idx_flat, offset_flat)


def kernel(score_map, offset_map):
    score3 = score_map.reshape(_BS, _NPTS, _FLAT)
    offset_flat = offset_map.reshape(_BS * _NCH * _FLAT)
    idx_tab = _tc_argmax(score3)
    out = _sc_gather(idx_tab.reshape(_BS * _NPTS_PAD * 128), offset_flat)
    return out.reshape(_BS, _PAD)[:, : _NCH].reshape(_BS, _NPTS, 2)


# speedup vs baseline: 2.5889x; 1.3410x over previous
import functools

import jax
import jax.numpy as jnp
from jax import lax
from jax.experimental import pallas as pl
from jax.experimental.pallas import tpu as pltpu
from jax.experimental.pallas import tpu_sc as plsc

_STRIDE = 4
_BS = 32
_NPTS = 17
_H = 128
_W = 128
_FLAT = _H * _W
_NCH = 2 * _NPTS
_PAD = 48


_ROWS = _BS * _NPTS
_BR = 32


def _tc_argmax_body(s_ref, o_ref):
    s = s_ref[...]
    m = jnp.max(s, axis=1, keepdims=True)
    iota = lax.broadcasted_iota(jnp.int32, (_BR, _FLAT), 1)
    cand = jnp.where(s == m, iota, jnp.int32(_FLAT))
    idx = jnp.min(cand, axis=1, keepdims=True)
    o_ref[...] = jnp.broadcast_to(idx, (_BR, 128))


def _tc_argmax(score_flat):
    return pl.pallas_call(
        _tc_argmax_body,
        grid=(_ROWS // _BR,),
        in_specs=[pl.BlockSpec((_BR, _FLAT), lambda i: (i, 0))],
        out_specs=pl.BlockSpec((_BR, 128), lambda i: (i, 0)),
        out_shape=jax.ShapeDtypeStruct((_ROWS, 128), jnp.int32),
    )(score_flat)


def _sc_gather_body(idx_hbm, offtab_hbm, out_hbm,
                    ptrs_v, iv_v, offidx_v, off_v, out_v, sem):
    b = lax.axis_index("s") * 2 + lax.axis_index("c")
    for base in (0, 16, 32):
        jv = lax.iota(jnp.int32, 16) + base
        ptrs_v[pl.ds(base, 16)] = (b * _NPTS + (jv >> 1)) * 128
    pltpu.async_copy(idx_hbm.at[ptrs_v], iv_v, sem).wait()
    for base in (0, 16, 32):
        jv = lax.iota(jnp.int32, 16) + base
        iv = iv_v[pl.ds(base, 16)]
        yv = iv >> 7
        xv = iv & (_W - 1)
        oidx = ((b * _NCH + jv) * _H + yv) * _W + xv
        offidx_v[pl.ds(base, 16)] = jnp.where(jv < _NCH, oidx, 0)
    pltpu.async_copy(offtab_hbm.at[offidx_v], off_v, sem).wait()
    for base in (0, 16, 32):
        jv = lax.iota(jnp.int32, 16) + base
        iv = iv_v[pl.ds(base, 16)]
        yv = iv >> 7
        xv = iv & (_W - 1)
        coarse = jnp.where((jv & 1) == 0, yv, xv).astype(jnp.float32)
        off = off_v[pl.ds(base, 16)]
        out_v[pl.ds(base, 16)] = (coarse + off) * float(_STRIDE)
    pltpu.sync_copy(out_v, out_hbm.at[pl.ds(b * _PAD, _PAD)])


def _sc_gather(idx_flat, offset_flat):
    mesh = plsc.VectorSubcoreMesh(core_axis_name="c", subcore_axis_name="s")
    f = functools.partial(
        pl.kernel,
        mesh=mesh,
        out_type=jax.ShapeDtypeStruct((_BS * _PAD,), jnp.float32),
        scratch_types=[
            pltpu.VMEM((_PAD,), jnp.int32),
            pltpu.VMEM((_PAD,), jnp.int32),
            pltpu.VMEM((_PAD,), jnp.int32),
            pltpu.VMEM((_PAD,), jnp.float32),
            pltpu.VMEM((_PAD,), jnp.float32),
            pltpu.SemaphoreType.DMA,
        ],
    )(_sc_gather_body)
    return f(idx_flat, offset_flat)


def kernel(score_map, offset_map):
    score_flat = score_map.reshape(_ROWS, _FLAT)
    offset_flat = offset_map.reshape(_BS * _NCH * _FLAT)
    idx_tab = _tc_argmax(score_flat)
    out = _sc_gather(idx_tab.reshape(_ROWS * 128), offset_flat)
    return out.reshape(_BS, _PAD)[:, : _NCH].reshape(_BS, _NPTS, 2)


# speedup vs baseline: 3.2375x; 1.2505x over previous
import functools

import jax
import jax.numpy as jnp
from jax import lax
from jax.experimental import pallas as pl
from jax.experimental.pallas import tpu as pltpu
from jax.experimental.pallas import tpu_sc as plsc

_STRIDE = 4
_BS = 32
_NPTS = 17
_H = 128
_W = 128
_FLAT = _H * _W
_NCH = 2 * _NPTS
_PAD = 48
_NCHUNK = _FLAT // 16
_UNROLL = 8


def _lane_shuffle(x, perm):
    return x.at[perm].get(mode="promise_in_bounds")


def _sc_full_body(score_hbm, off_hbm, out_hbm,
                  buf, offidx_v, off_v, out_v, sem0, sem1, semg):
    b = lax.axis_index("s") * 2 + lax.axis_index("c")
    base = b * (_NPTS * _FLAT)
    lane = lax.iota(jnp.int32, 16)
    sems = (sem0, sem1)
    handles = [None, None]
    handles[0] = pltpu.async_copy(
        score_hbm.at[pl.ds(base, _FLAT)], buf.at[0], sem0)
    ivs = [lane * 0, lane * 0, lane * 0]
    for r in range(_NPTS):
        if r + 1 < _NPTS:
            handles[(r + 1) % 2] = pltpu.async_copy(
                score_hbm.at[pl.ds(base + (r + 1) * _FLAT, _FLAT)],
                buf.at[(r + 1) % 2], sems[(r + 1) % 2])
        handles[r % 2].wait()
        bufr = buf.at[r % 2]

        def chunk_step(c, carry, bufr=bufr):
            rm, ch = carry
            for u in range(_UNROLL):
                i = c * _UNROLL + u
                v = bufr[pl.ds(i * 16, 16)]
                upd = v > rm
                rm = jnp.where(upd, v, rm)
                ch = jnp.where(upd, jnp.full((16,), i, jnp.int32), ch)
            return rm, ch

        rm0 = jnp.full((16,), -jnp.inf, jnp.float32)
        rm, ch = lax.fori_loop(0, _NCHUNK // _UNROLL, chunk_step,
                               (rm0, lane * 0))
        m = rm
        for sh in (8, 4, 2, 1):
            m = jnp.maximum(m, _lane_shuffle(m, lane ^ sh))
        flat = ch * 16 + lane
        cand = jnp.where(rm == m, flat, jnp.int32(_FLAT))
        for sh in (8, 4, 2, 1):
            cand = jnp.minimum(cand, _lane_shuffle(cand, lane ^ sh))
        for g in range(3):
            jv = lane + 16 * g
            ivs[g] = jnp.where((jv >> 1) == r, cand, ivs[g])
    for g in range(3):
        jv = lane + 16 * g
        iv = ivs[g]
        yv = iv >> 7
        xv = iv & (_W - 1)
        oidx = ((b * _NCH + jv) * _H + yv) * _W + xv
        offidx_v[pl.ds(16 * g, 16)] = jnp.where(jv < _NCH, oidx, 0)
    pltpu.async_copy(off_hbm.at[offidx_v], off_v, semg).wait()
    for g in range(3):
        jv = lane + 16 * g
        iv = ivs[g]
        yv = iv >> 7
        xv = iv & (_W - 1)
        coarse = jnp.where((jv & 1) == 0, yv, xv).astype(jnp.float32)
        off = off_v[pl.ds(16 * g, 16)]
        out_v[pl.ds(16 * g, 16)] = (coarse + off) * float(_STRIDE)
    pltpu.sync_copy(out_v, out_hbm.at[pl.ds(b * _PAD, _PAD)])


def _sc_full(score_flat, offset_flat):
    mesh = plsc.VectorSubcoreMesh(core_axis_name="c", subcore_axis_name="s")
    f = functools.partial(
        pl.kernel,
        mesh=mesh,
        out_type=jax.ShapeDtypeStruct((_BS * _PAD,), jnp.float32),
        scratch_types=[
            pltpu.VMEM((2, _FLAT), jnp.float32),
            pltpu.VMEM((_PAD,), jnp.int32),
            pltpu.VMEM((_PAD,), jnp.float32),
            pltpu.VMEM((_PAD,), jnp.float32),
            pltpu.SemaphoreType.DMA,
            pltpu.SemaphoreType.DMA,
            pltpu.SemaphoreType.DMA,
        ],
    )(_sc_full_body)
    return f(score_flat, offset_flat)


def kernel(score_map, offset_map):
    score_flat = score_map.reshape(_BS * _NPTS * _FLAT)
    offset_flat = offset_map.reshape(_BS * _NCH * _FLAT)
    out = _sc_full(score_flat, offset_flat)
    return out.reshape(_BS, _PAD)[:, : _NCH].reshape(_BS, _NPTS, 2)


# speedup vs baseline: 3.9174x; 1.2100x over previous
import functools

import jax
import jax.numpy as jnp
from jax import lax
from jax.experimental import pallas as pl
from jax.experimental.pallas import tpu as pltpu
from jax.experimental.pallas import tpu_sc as plsc

_STRIDE = 4
_BS = 32
_NPTS = 17
_H = 128
_W = 128
_FLAT = _H * _W
_NCH = 2 * _NPTS
_PAD = 48
_NCHUNK = _FLAT // 16
_UNROLL = 8


def _lane_shuffle(x, perm):
    return x.at[perm].get(mode="promise_in_bounds")


def _sc_full_body(score_hbm, off_hbm, out_hbm,
                  buf, offidx_v, off_v, out_v, sem0, sem1, semg):
    b = lax.axis_index("s") * 2 + lax.axis_index("c")
    base = b * (_NPTS * _FLAT)
    lane = lax.iota(jnp.int32, 16)
    sems = (sem0, sem1)
    handles = [None, None]
    handles[0] = pltpu.async_copy(
        score_hbm.at[pl.ds(base, _FLAT)], buf.at[0], sem0)
    ivs = [lane * 0, lane * 0, lane * 0]
    for r in range(_NPTS):
        if r + 1 < _NPTS:
            handles[(r + 1) % 2] = pltpu.async_copy(
                score_hbm.at[pl.ds(base + (r + 1) * _FLAT, _FLAT)],
                buf.at[(r + 1) % 2], sems[(r + 1) % 2])
        handles[r % 2].wait()
        bufr = buf.at[r % 2]

        def chunk_step(c, carry, bufr=bufr):
            rms, chs = carry
            cb = jnp.full((16,), c, jnp.int32)
            new_rms, new_chs = [], []
            for u in range(_UNROLL):
                v = bufr[pl.ds((c * _UNROLL + u) * 16, 16)]
                upd = v > rms[u]
                new_rms.append(jnp.where(upd, v, rms[u]))
                new_chs.append(jnp.where(upd, cb, chs[u]))
            return tuple(new_rms), tuple(new_chs)

        rm0 = jnp.full((16,), -jnp.inf, jnp.float32)
        rms, chs = lax.fori_loop(
            0, _NCHUNK // _UNROLL, chunk_step,
            ((rm0,) * _UNROLL, (lane * 0,) * _UNROLL))
        m = rms[0]
        for u in range(1, _UNROLL):
            m = jnp.maximum(m, rms[u])
        for sh in (8, 4, 2, 1):
            m = jnp.maximum(m, _lane_shuffle(m, lane ^ sh))
        cand = jnp.full((16,), _FLAT, jnp.int32)
        for u in range(_UNROLL):
            flat_u = chs[u] * (16 * _UNROLL) + (u * 16) + lane
            cand = jnp.minimum(cand,
                               jnp.where(rms[u] == m, flat_u, jnp.int32(_FLAT)))
        for sh in (8, 4, 2, 1):
            cand = jnp.minimum(cand, _lane_shuffle(cand, lane ^ sh))
        for g in range(3):
            jv = lane + 16 * g
            ivs[g] = jnp.where((jv >> 1) == r, cand, ivs[g])
    for g in range(3):
        jv = lane + 16 * g
        iv = ivs[g]
        yv = iv >> 7
        xv = iv & (_W - 1)
        oidx = ((b * _NCH + jv) * _H + yv) * _W + xv
        offidx_v[pl.ds(16 * g, 16)] = jnp.where(jv < _NCH, oidx, 0)
    pltpu.async_copy(off_hbm.at[offidx_v], off_v, semg).wait()
    for g in range(3):
        jv = lane + 16 * g
        iv = ivs[g]
        yv = iv >> 7
        xv = iv & (_W - 1)
        coarse = jnp.where((jv & 1) == 0, yv, xv).astype(jnp.float32)
        off = off_v[pl.ds(16 * g, 16)]
        out_v[pl.ds(16 * g, 16)] = (coarse + off) * float(_STRIDE)
    pltpu.sync_copy(out_v, out_hbm.at[pl.ds(b * _PAD, _PAD)])


def _sc_full(score_flat, offset_flat):
    mesh = plsc.VectorSubcoreMesh(core_axis_name="c", subcore_axis_name="s")
    f = functools.partial(
        pl.kernel,
        mesh=mesh,
        out_type=jax.ShapeDtypeStruct((_BS * _PAD,), jnp.float32),
        scratch_types=[
            pltpu.VMEM((2, _FLAT), jnp.float32),
            pltpu.VMEM((_PAD,), jnp.int32),
            pltpu.VMEM((_PAD,), jnp.float32),
            pltpu.VMEM((_PAD,), jnp.float32),
            pltpu.SemaphoreType.DMA,
            pltpu.SemaphoreType.DMA,
            pltpu.SemaphoreType.DMA,
        ],
    )(_sc_full_body)
    return f(score_flat, offset_flat)


def kernel(score_map, offset_map):
    score_flat = score_map.reshape(_BS * _NPTS * _FLAT)
    offset_flat = offset_map.reshape(_BS * _NCH * _FLAT)
    out = _sc_full(score_flat, offset_flat)
    return out.reshape(_BS, _PAD)[:, : _NCH].reshape(_BS, _NPTS, 2)


# speedup vs baseline: 3.9839x; 1.0170x over previous
import functools

import jax
import jax.numpy as jnp
from jax import lax
from jax.experimental import pallas as pl
from jax.experimental.pallas import tpu as pltpu
from jax.experimental.pallas import tpu_sc as plsc

_STRIDE = 4
_BS = 32
_NPTS = 17
_H = 128
_W = 128
_FLAT = _H * _W
_NCH = 2 * _NPTS
_PAD = 48
_NCHUNK = _FLAT // 16
_UNROLL = 8
_NBUF = 4


def _lane_shuffle(x, perm):
    return x.at[perm].get(mode="promise_in_bounds")


def _sc_full_body(score_hbm, off_hbm, out_hbm,
                  buf, offidx_v, off_v, out_v, sem0, sem1, sem2, sem3, semg):
    b = lax.axis_index("s") * 2 + lax.axis_index("c")
    base = b * (_NPTS * _FLAT)
    lane = lax.iota(jnp.int32, 16)
    sems = (sem0, sem1, sem2, sem3)
    handles = [None] * _NPTS
    for k in range(_NBUF):
        handles[k] = pltpu.async_copy(
            score_hbm.at[pl.ds(base + k * _FLAT, _FLAT)],
            buf.at[k], sems[k])
    ivs = [lane * 0, lane * 0, lane * 0]
    for r in range(_NPTS):
        handles[r].wait()
        bufr = buf.at[r % _NBUF]

        def chunk_step(c, carry, bufr=bufr):
            rms, chs = carry
            cb = jnp.full((16,), c, jnp.int32)
            new_rms, new_chs = [], []
            for u in range(_UNROLL):
                v = bufr[pl.ds((c * _UNROLL + u) * 16, 16)]
                upd = v > rms[u]
                new_rms.append(jnp.where(upd, v, rms[u]))
                new_chs.append(jnp.where(upd, cb, chs[u]))
            return tuple(new_rms), tuple(new_chs)

        rm0 = jnp.full((16,), -jnp.inf, jnp.float32)
        rms, chs = lax.fori_loop(
            0, _NCHUNK // _UNROLL, chunk_step,
            ((rm0,) * _UNROLL, (lane * 0,) * _UNROLL))
        m = rms[0]
        for u in range(1, _UNROLL):
            m = jnp.maximum(m, rms[u])
        for sh in (8, 4, 2, 1):
            m = jnp.maximum(m, _lane_shuffle(m, lane ^ sh))
        cand = jnp.full((16,), _FLAT, jnp.int32)
        for u in range(_UNROLL):
            flat_u = chs[u] * (16 * _UNROLL) + (u * 16) + lane
            cand = jnp.minimum(cand,
                               jnp.where(rms[u] == m, flat_u, jnp.int32(_FLAT)))
        for sh in (8, 4, 2, 1):
            cand = jnp.minimum(cand, _lane_shuffle(cand, lane ^ sh))
        for g in range(3):
            jv = lane + 16 * g
            ivs[g] = jnp.where((jv >> 1) == r, cand, ivs[g])
        if r + _NBUF < _NPTS:
            handles[r + _NBUF] = pltpu.async_copy(
                score_hbm.at[pl.ds(base + (r + _NBUF) * _FLAT, _FLAT)],
                buf.at[r % _NBUF], sems[r % _NBUF])
    for g in range(3):
        jv = lane + 16 * g
        iv = ivs[g]
        yv = iv >> 7
        xv = iv & (_W - 1)
        oidx = ((b * _NCH + jv) * _H + yv) * _W + xv
        offidx_v[pl.ds(16 * g, 16)] = jnp.where(jv < _NCH, oidx, 0)
    pltpu.async_copy(off_hbm.at[offidx_v], off_v, semg).wait()
    for g in range(3):
        jv = lane + 16 * g
        iv = ivs[g]
        yv = iv >> 7
        xv = iv & (_W - 1)
        coarse = jnp.where((jv & 1) == 0, yv, xv).astype(jnp.float32)
        off = off_v[pl.ds(16 * g, 16)]
        out_v[pl.ds(16 * g, 16)] = (coarse + off) * float(_STRIDE)
    pltpu.sync_copy(out_v, out_hbm.at[pl.ds(b * _PAD, _PAD)])


def _sc_full(score_flat, offset_flat):
    mesh = plsc.VectorSubcoreMesh(core_axis_name="c", subcore_axis_name="s")
    f = functools.partial(
        pl.kernel,
        mesh=mesh,
        out_type=jax.ShapeDtypeStruct((_BS * _PAD,), jnp.float32),
        scratch_types=[
            pltpu.VMEM((_NBUF, _FLAT), jnp.float32),
            pltpu.VMEM((_PAD,), jnp.int32),
            pltpu.VMEM((_PAD,), jnp.float32),
            pltpu.VMEM((_PAD,), jnp.float32),
            pltpu.SemaphoreType.DMA,
            pltpu.SemaphoreType.DMA,
            pltpu.SemaphoreType.DMA,
            pltpu.SemaphoreType.DMA,
            pltpu.SemaphoreType.DMA,
        ],
    )(_sc_full_body)
    return f(score_flat, offset_flat)


def kernel(score_map, offset_map):
    score_flat = score_map.reshape(_BS * _NPTS * _FLAT)
    offset_flat = offset_map.reshape(_BS * _NCH * _FLAT)
    out = _sc_full(score_flat, offset_flat)
    return out.reshape(_BS, _PAD)[:, : _NCH].reshape(_BS, _NPTS, 2)
